# baseline (speedup 1.0000x reference)
import functools

import jax
import jax.numpy as jnp
from jax import lax
from jax.experimental import pallas as pl
from jax.experimental.pallas import tpu as pltpu
from jax.experimental.pallas import tpu_sc as plsc

BATCH = 16384
HIST = 50
EMBED_DIM = 32
NUM_CORES = 2
NUM_SUBCORES = 16
NW = NUM_CORES * NUM_SUBCORES
BB = 128
NBB = BATCH // BB
BPW = NBB // NW
ROWS = BPW * BB
NITEM = HIST

_mesh = plsc.VectorSubcoreMesh(core_axis_name="c", subcore_axis_name="s")


@functools.partial(
    pl.kernel,
    mesh=_mesh,
    out_type=jax.ShapeDtypeStruct((BATCH * HIST,), jnp.int32),
    scratch_types=[
        pltpu.VMEM((7, 8, 512), jnp.int32),
        pltpu.SemaphoreType.DMA((7,)),
        pltpu.SemaphoreType.DMA((7,)),
    ],
    compiler_params=pltpu.CompilerParams(
        use_tc_tiling_on_sc=True, needs_layout_passes=False
    ),
)
def _x_untile(xt_hbm, xl_hbm, vbuf, rsem, wsem):
    wid = lax.axis_index("s") * NUM_CORES + lax.axis_index("c")
    col0 = wid * 512

    def rd(hb):
        rows = 8 if hb < 6 else 2
        return pltpu.make_async_copy(
            xt_hbm.at[pl.ds(hb * 8, rows), pl.ds(col0, 512)],
            vbuf.at[hb, pl.ds(0, rows)],
            rsem.at[hb],
        )

    def wr(h):
        hb, r = divmod(h, 8)
        return pltpu.make_async_copy(
            vbuf.at[hb, r],
            xl_hbm.at[pl.ds(h * BATCH + col0, 512)],
            wsem.at[hb],
        )

    for hb in range(7):
        rd(hb).start()
    for hb in range(7):
        rd(hb).wait()
        rows = 8 if hb < 6 else 2
        for r in range(rows):
            wr(hb * 8 + r).start()
    for h in range(HIST):
        wr(h).wait()


@functools.partial(
    pl.kernel,
    mesh=_mesh,
    out_type=jax.ShapeDtypeStruct((HIST, 4, NBB, 8, BB), jnp.float32),
    scratch_types=[
        pltpu.VMEM((2, ROWS), jnp.int32),
        pltpu.VMEM((2, ROWS, EMBED_DIM), jnp.float32),
        pltpu.VMEM((2, 16, 10, 129), jnp.float32),
        pltpu.SemaphoreType.DMA((2,)),
        pltpu.SemaphoreType.DMA((2,)),
        pltpu.SemaphoreType.DMA((2,)),
    ],
    compiler_params=pltpu.CompilerParams(
        use_tc_tiling_on_sc=False, needs_layout_passes=False
    ),
)
def _emb_lookup(xl_hbm, table_hbm, out_hbm, idx_v, gbuf, obuf, isem, gsem, wsem):
    wid = lax.axis_index("s") * NUM_CORES + lax.axis_index("c")
    col0 = wid * ROWS
    bb0 = wid * BPW

    def idx_copy(t, b):
        return pltpu.async_copy(
            xl_hbm.at[pl.ds(t * BATCH + col0, ROWS)], idx_v.at[b], isem.at[b]
        )

    def gather_copy(b):
        return pltpu.async_copy(table_hbm.at[idx_v.at[b]], gbuf.at[b], gsem.at[b])

    def write_copies(t, b, do_issue):
        for cb in range(4):
            cp = pltpu.make_async_copy(
                obuf.at[b, pl.ds(cb * BPW, BPW), pl.ds(0, 8), pl.ds(0, BB)],
                out_hbm.at[t, cb, pl.ds(bb0, BPW)],
                wsem.at[b],
            )
            if do_issue:
                cp.start()
            else:
                cp.wait()

    lanes = lax.iota(jnp.int32, 16)
    cb4_lo = (lanes // 8) * BPW
    cb4_hi = ((lanes + 16) // 8) * BPW
    ci_vec = lax.rem(lanes, 8)

    def transpose_item(b):
        src = gbuf.at[b]
        dst = obuf.at[b]

        def blk_body(rb, _):
            for u in range(8):
                r = rb * 8 + u
                j = r // BB
                bi = lax.rem(r, BB)
                ja = jnp.full((16,), j, jnp.int32)
                bia = jnp.full((16,), bi, jnp.int32)
                v_lo = src[r, pl.ds(0, 16)]
                v_hi = src[r, pl.ds(16, 16)]
                plsc.store_scatter(dst, [cb4_lo + ja, ci_vec, bia], v_lo)
                plsc.store_scatter(dst, [cb4_hi + ja, ci_vec, bia], v_hi)
            return 0

        lax.fori_loop(0, ROWS // 8, blk_body, 0)

    idx_copy(0, 0).wait()
    gather_copy(0)
    idx_copy(1, 1)

    def body(t, _):
        b = lax.rem(t, 2)
        nb = lax.rem(t + 1, 2)

        pltpu.make_async_copy(
            table_hbm.at[idx_v.at[b]], gbuf.at[b], gsem.at[b]
        ).wait()

        @pl.when(t < NITEM - 2)
        def _():
            idx_copy(t + 2, b)

        @pl.when(t < NITEM - 1)
        def _():
            pltpu.make_async_copy(
                xl_hbm.at[pl.ds((t + 1) * BATCH + col0, ROWS)],
                idx_v.at[nb],
                isem.at[nb],
            ).wait()
            gather_copy(nb)

        @pl.when(t >= 2)
        def _():
            write_copies(t - 2, b, do_issue=False)

        transpose_item(b)
        write_copies(t, b, do_issue=True)
        return 0

    lax.fori_loop(0, NITEM, body, 0)
    write_copies(NITEM - 2, (NITEM - 2) % 2, do_issue=False)
    write_copies(NITEM - 1, (NITEM - 1) % 2, do_issue=False)


def kernel(x, table):
    xt = x.T.astype(jnp.int32)
    xl = _x_untile(xt)
    out5 = _emb_lookup(xl, table)
    return out5.transpose(2, 4, 0, 1, 3).reshape(BATCH, HIST, EMBED_DIM)
